# Initial kernel scaffold; baseline (speedup 1.0000x reference)
import jax
import jax.numpy as jnp
from jax.experimental import pallas as pl


def kernel(data, emb_table):
    raise NotImplementedError("write your pallas kernel here")


# speedup vs baseline: 6.4597x; 6.4597x over previous
import functools

import jax
import jax.numpy as jnp
from jax.experimental import pallas as pl
from jax.experimental.pallas import tpu as pltpu

_BN = 1000


def _concat_body(data_ref, emb_ref, out_ref):
    bn = data_ref.shape[0]
    emb = jnp.broadcast_to(emb_ref[...][None, :, :], (bn,) + emb_ref.shape)
    out_ref[...] = jnp.concatenate([data_ref[...], emb], axis=-1)


@jax.jit
def kernel(data, emb_table):
    g, n, t, f = data.shape
    e = emb_table.shape[1]
    rows = g * n
    d2 = data.reshape(rows, t, f)
    grid = rows // _BN
    out = pl.pallas_call(
        _concat_body,
        grid=(grid,),
        in_specs=[
            pl.BlockSpec((_BN, t, f), lambda i: (i, 0, 0)),
            pl.BlockSpec((t, e), lambda i: (0, 0)),
        ],
        out_specs=pl.BlockSpec((_BN, t, f + e), lambda i: (i, 0, 0)),
        out_shape=jax.ShapeDtypeStruct((rows, t, f + e), data.dtype),
    )(d2, emb_table)
    return out.reshape(g, n, t, f + e)


# speedup vs baseline: 15.4213x; 2.3873x over previous
import jax
import jax.numpy as jnp
from jax.experimental import pallas as pl

_BN = 1000


def _concat_body(data_ref, emb_ref, out_ref):
    bn, t, e = data_ref.shape[1], emb_ref.shape[0], emb_ref.shape[1]
    emb = jnp.broadcast_to(emb_ref[...][None, None, :, :], (1, bn, t, e))
    out_ref[...] = jnp.concatenate([data_ref[...], emb], axis=-1)


@jax.jit
def kernel(data, emb_table):
    g, n, t, f = data.shape
    e = emb_table.shape[1]
    return pl.pallas_call(
        _concat_body,
        grid=(g, n // _BN),
        in_specs=[
            pl.BlockSpec((1, _BN, t, f), lambda i, j: (i, j, 0, 0)),
            pl.BlockSpec((t, e), lambda i, j: (0, 0)),
        ],
        out_specs=pl.BlockSpec((1, _BN, t, f + e), lambda i, j: (i, j, 0, 0)),
        out_shape=jax.ShapeDtypeStruct((g, n, t, f + e), data.dtype),
    )(data, emb_table)


# speedup vs baseline: 32.5431x; 2.1103x over previous
import jax
import jax.numpy as jnp
from jax.experimental import pallas as pl

_BN = 512


def _body(dt_ref, emb_ref, out_ref):
    row = emb_ref[pl.ds(pl.program_id(0), 1), :]
    emb = jnp.broadcast_to(row, (128, 13))
    for g in range(out_ref.shape[0]):
        for k in range(_BN // 128):
            x = dt_ref[0, :, g, pl.ds(k * 128, 128)]
            y = x.T
            out_ref[g, 0, pl.ds(k * 128, 128), :] = jnp.concatenate(
                [y, emb], axis=-1)


@jax.jit
def kernel(data, emb_table):
    g, n, t, f = data.shape
    e = emb_table.shape[1]
    dt = jnp.transpose(data, (2, 3, 0, 1))
    out_t = pl.pallas_call(
        _body,
        grid=(t, (n + _BN - 1) // _BN),
        in_specs=[
            pl.BlockSpec((1, f, g, _BN), lambda i, j: (i, 0, 0, j)),
            pl.BlockSpec((t, e), lambda i, j: (0, 0)),
        ],
        out_specs=pl.BlockSpec((g, 1, _BN, f + e), lambda i, j: (0, i, j, 0)),
        out_shape=jax.ShapeDtypeStruct((g, t, n, f + e), data.dtype),
    )(dt, emb_table)
    return jnp.transpose(out_t, (0, 2, 1, 3))


# speedup vs baseline: 41.9240x; 1.2883x over previous
import jax
import jax.numpy as jnp
from jax.experimental import pallas as pl

_BN = 1024


def _body(dt_ref, emb_ref, out_ref):
    row = emb_ref[pl.ds(pl.program_id(0), 1), :]
    emb = jnp.broadcast_to(row, (128, 13))
    for g in range(out_ref.shape[0]):
        for k in range(_BN // 128):
            x = dt_ref[0, :, g, pl.ds(k * 128, 128)]
            y = x.T
            out_ref[g, 0, pl.ds(k * 128, 128), :] = jnp.concatenate(
                [y, emb], axis=-1)


@jax.jit
def kernel(data, emb_table):
    g, n, t, f = data.shape
    e = emb_table.shape[1]
    dt = jnp.transpose(data, (2, 3, 0, 1))
    out_t = pl.pallas_call(
        _body,
        grid=(t, (n + _BN - 1) // _BN),
        in_specs=[
            pl.BlockSpec((1, f, g, _BN), lambda i, j: (i, 0, 0, j)),
            pl.BlockSpec((t, e), lambda i, j: (0, 0)),
        ],
        out_specs=pl.BlockSpec((g, 1, _BN, f + e), lambda i, j: (0, i, j, 0)),
        out_shape=jax.ShapeDtypeStruct((g, t, n, f + e), data.dtype),
    )(dt, emb_table)
    return jnp.transpose(out_t, (0, 2, 1, 3))


# speedup vs baseline: 51.1863x; 1.2209x over previous
import jax
import jax.numpy as jnp
from jax.experimental import pallas as pl

_BN = 2048


def _body(dt_ref, emb_ref, out_ref):
    row = emb_ref[pl.ds(pl.program_id(0), 1), :]
    emb = jnp.broadcast_to(row, (128, 13))
    for g in range(out_ref.shape[0]):
        for k in range(_BN // 128):
            x = dt_ref[0, :, g, pl.ds(k * 128, 128)]
            y = x.T
            out_ref[g, 0, pl.ds(k * 128, 128), :] = jnp.concatenate(
                [y, emb], axis=-1)


@jax.jit
def kernel(data, emb_table):
    g, n, t, f = data.shape
    e = emb_table.shape[1]
    dt = jnp.transpose(data, (2, 3, 0, 1))
    out_t = pl.pallas_call(
        _body,
        grid=(t, (n + _BN - 1) // _BN),
        in_specs=[
            pl.BlockSpec((1, f, g, _BN), lambda i, j: (i, 0, 0, j)),
            pl.BlockSpec((t, e), lambda i, j: (0, 0)),
        ],
        out_specs=pl.BlockSpec((g, 1, _BN, f + e), lambda i, j: (0, i, j, 0)),
        out_shape=jax.ShapeDtypeStruct((g, t, n, f + e), data.dtype),
    )(dt, emb_table)
    return jnp.transpose(out_t, (0, 2, 1, 3))
